# baseline (speedup 1.0000x reference)
import functools

import jax
import jax.numpy as jnp
from jax import lax
from jax.experimental import pallas as pl
from jax.experimental.pallas import tpu as pltpu
from jax.experimental.pallas import tpu_sc as plsc

MAX_YR_LEN = 120
T = 12 * MAX_YR_LEN
BATCH = 16384
L = 16

NC = 2
NS = 16
NW = NC * NS
B_PER_W = BATCH // NW
CHUNK = 32
NCHUNK = B_PER_W // CHUNK


def _prep_body(qx_ref, sex_ref, age_ref, ann_ref, table_ref, key_ref):
    qm = jnp.exp(jnp.log(qx_ref[...] + 1.0) * (1.0 / 12.0)) - 1.0
    shift = 12 - 11 * ann_ref[0]
    j = lax.broadcasted_iota(jnp.int32, (MAX_YR_LEN, 2 * T), 1)
    y = lax.broadcasted_iota(jnp.int32, (MAX_YR_LEN, 2 * T), 0)
    a = j // T
    pos = (j % T) + a * shift
    e = jnp.where((pos // 12 == y) & (pos < T), 1.0, 0.0).astype(jnp.float32)
    table_ref[...] = lax.dot_general(
        qm, e, (((1,), (0,)), ((), ())), preferred_element_type=jnp.float32
    )
    key_ref[...] = sex_ref[...] * 2 + age_ref[...]


def _prep(qx, sex2d, age2d, ann):
    return pl.pallas_call(
        _prep_body,
        out_shape=[
            jax.ShapeDtypeStruct((2, 2 * T), jnp.float32),
            jax.ShapeDtypeStruct(sex2d.shape, jnp.int32),
        ],
        in_specs=[
            pl.BlockSpec(memory_space=pltpu.VMEM),
            pl.BlockSpec(memory_space=pltpu.VMEM),
            pl.BlockSpec(memory_space=pltpu.VMEM),
            pl.BlockSpec(memory_space=pltpu.SMEM),
        ],
        out_specs=[
            pl.BlockSpec(memory_space=pltpu.VMEM),
            pl.BlockSpec(memory_space=pltpu.VMEM),
        ],
    )(qx, sex2d, age2d, ann)


def _sc_body(nrows, table_hbm, key_hbm, out_hbm, table_v, idx_v, buf_v, ssem0, ssem1):
    b_per_w = nrows // NW
    nchunk = b_per_w // CHUNK
    wid = lax.axis_index("s") * NC + lax.axis_index("c")
    base = wid * b_per_w
    pltpu.sync_copy(table_hbm, table_v)
    pltpu.sync_copy(key_hbm.at[pl.ds(base, b_per_w)], idx_v)

    ssems = (ssem0, ssem1)

    def build(g, slot):
        for q in range(CHUNK // L):
            kvec = idx_v[pl.ds(g * CHUNK + q * L, L)]
            bases = [kvec[r] * T for r in range(L)]

            def col_body(i, _, q=q, bases=bases):
                c = i * L
                vs = [table_v[pl.ds(bases[r] + c, L)] for r in range(L)]
                for r in range(L):
                    buf_v[slot, q * L + r, pl.ds(c, L)] = vs[r]
                return 0

            lax.fori_loop(0, T // L, col_body, 0, unroll=2)

    def wait_scatter(slot):
        pltpu.make_async_copy(
            buf_v.at[slot], out_hbm.at[pl.ds(base, CHUNK)], ssems[slot]
        ).wait()

    def pair_body(p, _):
        for slot in range(2):
            g = 2 * p + slot

            @pl.when(p > 0)
            def _():
                wait_scatter(slot)

            build(g, slot)
            pltpu.async_copy(
                buf_v.at[slot],
                out_hbm.at[pl.ds(base + g * CHUNK, CHUNK)],
                ssems[slot],
            )
        return 0

    lax.fori_loop(0, nchunk // 2, pair_body, 0)
    wait_scatter(0)
    wait_scatter(1)


@functools.cache
def _sc_lookup(nrows):
    return pl.kernel(
        functools.partial(_sc_body, nrows),
        out_type=jax.ShapeDtypeStruct((nrows, T), jnp.float32),
        mesh=plsc.VectorSubcoreMesh(
            core_axis_name="c", subcore_axis_name="s", num_cores=NC, num_subcores=NS
        ),
        scratch_types=[
            pltpu.VMEM((4 * T,), jnp.float32),
            pltpu.VMEM((nrows // NW,), jnp.int32),
            pltpu.VMEM((2, CHUNK, T), jnp.float32),
            pltpu.SemaphoreType.DMA,
            pltpu.SemaphoreType.DMA,
        ],
        compiler_params=pltpu.CompilerParams(
            needs_layout_passes=False, use_tc_tiling_on_sc=True
        ),
    )


def kernel(mp_idx, qx, annual):
    ann = jnp.asarray(annual, jnp.int32).reshape(1)
    side = 128
    sex2d = mp_idx[:, 0].reshape(side, side)
    age2d = mp_idx[:, 1].reshape(side, side)
    table2, key2 = _prep(qx, sex2d, age2d, ann)
    table = table2.reshape(4 * T)
    key = key2.reshape(BATCH)
    half = BATCH // 2
    lookup = _sc_lookup(half)
    o1 = lookup(table, key[:half])
    o2 = lookup(table, key[half:])
    return jnp.concatenate([o1, o2], axis=0)


# speedup vs baseline: 1.3532x; 1.3532x over previous
import functools

import jax
import jax.numpy as jnp
from jax import lax
from jax.experimental import pallas as pl
from jax.experimental.pallas import tpu as pltpu
from jax.experimental.pallas import tpu_sc as plsc

MAX_YR_LEN = 120
T = 12 * MAX_YR_LEN
BATCH = 16384
L = 16

NC = 2
NS = 16
NW = NC * NS
B_PER_W = BATCH // NW
CHUNK = 32
NCHUNK = B_PER_W // CHUNK


def _prep_body(qx_ref, sex_ref, age_ref, ann_ref, table_ref, key_ref):
    qm = jnp.exp(jnp.log(qx_ref[...] + 1.0) * (1.0 / 12.0)) - 1.0
    shift = 12 - 11 * ann_ref[0]
    j = lax.broadcasted_iota(jnp.int32, (MAX_YR_LEN, 2 * T), 1)
    y = lax.broadcasted_iota(jnp.int32, (MAX_YR_LEN, 2 * T), 0)
    a = j // T
    pos = (j % T) + a * shift
    e = jnp.where((pos // 12 == y) & (pos < T), 1.0, 0.0).astype(jnp.float32)
    table_ref[...] = lax.dot_general(
        qm, e, (((1,), (0,)), ((), ())), preferred_element_type=jnp.float32
    )
    key_ref[...] = sex_ref[...] * 2 + age_ref[...]


def _prep(qx, sex2d, age2d, ann):
    return pl.pallas_call(
        _prep_body,
        out_shape=[
            jax.ShapeDtypeStruct((2, 2 * T), jnp.float32),
            jax.ShapeDtypeStruct(sex2d.shape, jnp.int32),
        ],
        in_specs=[
            pl.BlockSpec(memory_space=pltpu.VMEM),
            pl.BlockSpec(memory_space=pltpu.VMEM),
            pl.BlockSpec(memory_space=pltpu.VMEM),
            pl.BlockSpec(memory_space=pltpu.SMEM),
        ],
        out_specs=[
            pl.BlockSpec(memory_space=pltpu.VMEM),
            pl.BlockSpec(memory_space=pltpu.VMEM),
        ],
    )(qx, sex2d, age2d, ann)


def _sc_body(table_hbm, key_hbm, out_hbm, table_v, idx_v, buf_v, ssem0, ssem1):
    wid = lax.axis_index("s") * NC + lax.axis_index("c")
    base = wid * B_PER_W
    pltpu.sync_copy(table_hbm, table_v)
    pltpu.sync_copy(key_hbm.at[pl.ds(base, B_PER_W)], idx_v)

    ssems = (ssem0, ssem1)

    def build(g, slot):
        for q in range(CHUNK // L):
            kvec = idx_v[pl.ds(g * CHUNK + q * L, L)]
            bases = [kvec[r] * T for r in range(L)]

            def col_body(i, _, q=q, bases=bases):
                c = i * L
                vs = [table_v[pl.ds(bases[r] + c, L)] for r in range(L)]
                for r in range(L):
                    buf_v[slot, q * L + r, pl.ds(c, L)] = vs[r]
                return 0

            lax.fori_loop(0, T // L, col_body, 0, unroll=2)

    def wait_scatter(slot):
        pltpu.make_async_copy(
            buf_v.at[slot], out_hbm.at[pl.ds(base, CHUNK)], ssems[slot]
        ).wait()

    def pair_body(p, _):
        for slot in range(2):
            g = 2 * p + slot

            @pl.when(p > 0)
            def _():
                wait_scatter(slot)

            build(g, slot)
            pltpu.async_copy(
                buf_v.at[slot],
                out_hbm.at[pl.ds(base + g * CHUNK, CHUNK)],
                ssems[slot],
            )
        return 0

    lax.fori_loop(0, NCHUNK // 2, pair_body, 0)
    wait_scatter(0)
    wait_scatter(1)


@functools.cache
def _sc_lookup():
    return pl.kernel(
        _sc_body,
        out_type=jax.ShapeDtypeStruct((BATCH, T), jnp.float32),
        mesh=plsc.VectorSubcoreMesh(
            core_axis_name="c", subcore_axis_name="s", num_cores=NC, num_subcores=NS
        ),
        scratch_types=[
            pltpu.VMEM((4 * T,), jnp.float32),
            pltpu.VMEM((B_PER_W,), jnp.int32),
            pltpu.VMEM((2, CHUNK, T), jnp.float32),
            pltpu.SemaphoreType.DMA,
            pltpu.SemaphoreType.DMA,
        ],
        compiler_params=pltpu.CompilerParams(
            needs_layout_passes=False, use_tc_tiling_on_sc=True
        ),
    )


def kernel(mp_idx, qx, annual):
    ann = jnp.asarray(annual, jnp.int32).reshape(1)
    side = 128
    sex2d = mp_idx[:, 0].reshape(side, side)
    age2d = mp_idx[:, 1].reshape(side, side)
    table2, key2 = _prep(qx, sex2d, age2d, ann)
    table = table2.reshape(4 * T)
    key = key2.reshape(BATCH)
    return _sc_lookup()(table, key)
